# baseline (speedup 1.0000x reference)
import functools

import jax
import jax.numpy as jnp
from jax import lax
from jax.experimental import pallas as pl
from jax.experimental.pallas import tpu as pltpu
from jax.experimental.pallas import tpu_sc as plsc

_NUM_CORES = 2
_NUM_SUBCORES = 16
_NUM_WORKERS = _NUM_CORES * _NUM_SUBCORES


def _split_rows(rows_per_sc: int, b: int, d: int) -> int:
    best_r, best_t = 0, None
    n_stream = _NUM_SUBCORES - 1
    for r in range(0, rows_per_sc // n_stream + 1, 8):
        sp = rows_per_sc - n_stream * r
        if sp * d * 4 > 6 * 1024 * 1024:
            continue
        t = max(r / 96.0, sp / 900.0)
        if best_t is None or t < best_t:
            best_r, best_t = r, t
    return best_r


@functools.lru_cache(maxsize=None)
def _make_bcast_rows(b: int, p: int, d: int):
    rows_per_sc = p // _NUM_CORES
    rows_per_w = _split_rows(rows_per_sc, b, d)
    sp_rows = rows_per_sc - (_NUM_SUBCORES - 1) * rows_per_w
    budget_rows = (7 * 1024 * 1024 + 512 * 1024) // (d * 4)
    chunk = min(rows_per_w, max(8, (budget_rows - sp_rows) // _NUM_SUBCORES))
    chunk -= chunk % 8
    n_chunks = rows_per_w // chunk if chunk else 0
    tail = rows_per_w - n_chunks * chunk

    mesh = plsc.VectorSubcoreMesh(core_axis_name="c", subcore_axis_name="s")

    @functools.partial(
        pl.kernel,
        out_type=jax.ShapeDtypeStruct((b, p, d), jnp.float32),
        mesh=mesh,
        scratch_types=[
            pltpu.VMEM((max(chunk, 1), d), jnp.float32),
            pltpu.VMEM_SHARED((max(sp_rows, 1), d), jnp.float32),
            pltpu.SemaphoreType.DMA,
            pltpu.SemaphoreType.DMA,
            pltpu.SemaphoreType.DMA,
        ],
    )
    def bcast_rows(wpos_hbm, out_hbm, buf, spbuf, rsem, wsem, spsem):
        cid = lax.axis_index("c")
        sid = lax.axis_index("s")
        sc_base = cid * rows_per_sc

        def stream_span(r0, n):
            pltpu.make_async_copy(
                wpos_hbm.at[pl.ds(r0, n)], buf.at[pl.ds(0, n)], rsem).start()
            pltpu.make_async_copy(
                wpos_hbm.at[pl.ds(r0, n)], buf.at[pl.ds(0, n)], rsem).wait()
            whs = [
                pltpu.async_copy(
                    buf.at[pl.ds(0, n)], out_hbm.at[bi, pl.ds(r0, n)], wsem)
                for bi in range(b)
            ]
            for h in whs:
                h.wait()

        if sp_rows:
            sp_base = sc_base + (_NUM_SUBCORES - 1) * rows_per_w

            @pl.when(sid == 0)
            def _sp_channel():
                pltpu.async_copy(
                    wpos_hbm.at[pl.ds(sp_base, sp_rows)], spbuf, spsem)
                pltpu.make_async_copy(
                    wpos_hbm.at[pl.ds(sp_base, sp_rows)], spbuf, spsem).wait()
                whs = [
                    pltpu.async_copy(
                        spbuf, out_hbm.at[bi, pl.ds(sp_base, sp_rows)], spsem)
                    for bi in range(b)
                ]
                for h in whs:
                    h.wait()

        if rows_per_w:
            @pl.when(sid > 0)
            def _stream_channel():
                base = sc_base + (sid - 1) * rows_per_w
                for i in range(n_chunks):
                    stream_span(base + i * chunk, chunk)
                if tail:
                    stream_span(base + n_chunks * chunk, tail)

    return bcast_rows


def kernel(tokens, W_pos):
    b, p = tokens.shape
    d = W_pos.shape[1]
    return _make_bcast_rows(b, p, d)(W_pos)


# speedup vs baseline: 1.0275x; 1.0275x over previous
import functools

import jax
import jax.numpy as jnp
from jax import lax
from jax.experimental import pallas as pl
from jax.experimental.pallas import tpu as pltpu
from jax.experimental.pallas import tpu_sc as plsc

_NUM_CORES = 2
_NUM_SUBCORES = 16
_NUM_WORKERS = _NUM_CORES * _NUM_SUBCORES


@functools.lru_cache(maxsize=None)
def _make_bcast_rows(b: int, p: int, d: int):
    rows_per_w = p // _NUM_WORKERS
    chunk = rows_per_w
    while _NUM_SUBCORES * chunk * d * 4 > 7 * 1024 * 1024 + 512 * 1024:
        chunk //= 2
    chunk = max(8, chunk - chunk % 8)
    n_chunks = rows_per_w // chunk
    tail = rows_per_w - n_chunks * chunk

    mesh = plsc.VectorSubcoreMesh(core_axis_name="c", subcore_axis_name="s")

    @functools.partial(
        pl.kernel,
        out_type=jax.ShapeDtypeStruct((b, p, d), jnp.float32),
        mesh=mesh,
        scratch_types=[pltpu.VMEM((chunk, d), jnp.float32)],
    )
    def bcast_rows(wpos_hbm, out_hbm, buf):
        wid = lax.axis_index("s") * _NUM_CORES + lax.axis_index("c")
        base = wid * rows_per_w

        def span(r0, n):
            pltpu.sync_copy(wpos_hbm.at[pl.ds(r0, n)], buf.at[pl.ds(0, n)])
            for bi in range(b):
                pltpu.sync_copy(
                    buf.at[pl.ds(0, n)], out_hbm.at[bi, pl.ds(r0, n)])

        for i in range(n_chunks):
            span(base + i * chunk, chunk)
        if tail:
            span(base + n_chunks * chunk, tail)

    return bcast_rows


def kernel(tokens, W_pos):
    b, p = tokens.shape
    d = W_pos.shape[1]
    return _make_bcast_rows(b, p, d)(W_pos)
